# Initial kernel scaffold; baseline (speedup 1.0000x reference)
import jax
import jax.numpy as jnp
from jax.experimental import pallas as pl


def kernel(ids, feats, adj, W_self1, W_neib1, W_self2, W_neib2, fc_W, fc_b):
    raise NotImplementedError("write your pallas kernel here")


# speedup vs baseline: 4.9483x; 4.9483x over previous
import jax
import jax.numpy as jnp
import numpy as np
from jax import lax
from jax.experimental import pallas as pl
from jax.experimental.pallas import tpu as pltpu
from jax.experimental.pallas import tpu_sc as plsc

_N_NODES = 100000
_D = 128
_DEG = 32
_B = 1024
_NS1 = 25
_NS2 = 10
_HID = 128
_NW = 32
_SEEDS_W = _B // _NW
_CH = 80
_C1 = (_B * _NS1) // (_NW * _CH)
_C2 = (_B * _NS1 * _NS2) // (_NW * _CH)
_G1_W = _C1 * _CH
_G2_W = _C2 * _CH


def _rotl32(x, d):
    d = np.uint32(d)
    return (x << d) | (x >> np.uint32(32 - d))


def _tf2x32(k1, k2, c1, c2):
    rot = ((13, 15, 26, 6), (17, 29, 16, 24))
    ks = (k1, k2, k1 ^ k2 ^ np.uint32(0x1BD11BDA))
    x0 = (c1 + ks[0]).astype(np.uint32)
    x1 = (c2 + ks[1]).astype(np.uint32)
    for i in range(5):
        for r in rot[i % 2]:
            x0 = (x0 + x1).astype(np.uint32)
            x1 = _rotl32(x1, r)
            x1 = x0 ^ x1
        x0 = (x0 + ks[(i + 1) % 3]).astype(np.uint32)
        x1 = (x1 + ks[(i + 2) % 3] + np.uint32(i + 1)).astype(np.uint32)
    return x0, x1


def _sel_draw(layer, n):
    key = (np.zeros(1, np.uint32), np.full(1, 42, np.uint32))
    key = _tf2x32(key[0], key[1], np.zeros(1, np.uint32),
                  np.full(1, layer, np.uint32))
    b1, b2 = _tf2x32(key[0], key[1], np.zeros(2, np.uint32),
                     np.arange(2, dtype=np.uint32))
    k_hi = (b1[0:1], b2[0:1])
    k_lo = (b1[1:2], b2[1:2])
    idx = np.arange(n, dtype=np.uint64)
    c1 = (idx >> np.uint64(32)).astype(np.uint32)
    c2 = (idx & np.uint64(0xFFFFFFFF)).astype(np.uint32)
    h1, h2 = _tf2x32(k_hi[0], k_hi[1], c1, c2)
    l1, l2 = _tf2x32(k_lo[0], k_lo[1], c1, c2)
    hi = h1 ^ h2
    lo = l1 ^ l2
    return ((hi % np.uint32(_DEG)) * np.uint32(0)
            + (lo % np.uint32(_DEG))).astype(np.int32)


_SEL1 = _sel_draw(0, _B * _NS1).reshape(_NW, _C1, _CH)
_SEL2 = _sel_draw(1, _B * _NS1 * _NS2).reshape(_NW, _C2, _CH)
_R1 = (np.arange(_B * _NS1, dtype=np.int32) // _NS1).reshape(_NW, _C1, _CH)
_R2 = (np.arange(_B * _NS1 * _NS2, dtype=np.int32) // _NS2).reshape(
    _NW, _C2, _CH)


def _sc_body(ids_h, feats_h, adjf_h, r1_h, s1_h, r2_h, s2_h,
             f0_h, f1_h, sum2_h, cur1_h,
             ids_v, bufa1, bufb1, cur1_v, bufa2, bufb2, cur2_v,
             fbuf, accbuf, gsem, fsem):
    c = lax.axis_index("c")
    s = lax.axis_index("s")
    w = s * 2 + c
    seed0 = w * _SEEDS_W
    g0 = w * _G1_W
    q0 = w * _G2_W

    pltpu.sync_copy(r1_h.at[w], bufa1)

    def gi1(k, carry):
        pltpu.async_copy(ids_h.at[bufa1.at[k]], bufb1.at[k], gsem)
        return carry

    def di1(k, carry):
        pltpu.make_async_copy(ids_h.at[bufa1.at[k]], bufb1.at[k], gsem).wait()
        return carry

    lax.fori_loop(0, _C1, gi1, 0)
    lax.fori_loop(0, _C1, di1, 0)
    pltpu.sync_copy(s1_h.at[w], bufa1)

    def ma1(r, carry):
        for seg in range(_CH // 16):
            x = bufb1[r, pl.ds(seg * 16, 16)]
            y = bufa1[r, pl.ds(seg * 16, 16)]
            bufb1[r, pl.ds(seg * 16, 16)] = x * _DEG + y
        return carry

    lax.fori_loop(0, _C1, ma1, 0)

    def gc1(k, carry):
        pltpu.async_copy(adjf_h.at[bufb1.at[k]], cur1_v.at[k], gsem)
        return carry

    def dc1(k, carry):
        pltpu.make_async_copy(adjf_h.at[bufb1.at[k]], cur1_v.at[k],
                              gsem).wait()
        return carry

    lax.fori_loop(0, _C1, gc1, 0)
    lax.fori_loop(0, _C1, dc1, 0)

    def wc1(k, carry):
        pltpu.sync_copy(cur1_v.at[k], cur1_h.at[pl.ds(g0 + k * _CH, _CH)])
        return carry

    lax.fori_loop(0, _C1, wc1, 0)

    pltpu.sync_copy(ids_h.at[pl.ds(seed0, _SEEDS_W)], ids_v)
    pltpu.async_copy(feats_h.at[ids_v], fbuf.at[pl.ds(0, _SEEDS_W)],
                     fsem).wait()
    pltpu.sync_copy(fbuf.at[pl.ds(0, _SEEDS_W)],
                    f0_h.at[pl.ds(seed0, _SEEDS_W)])

    pltpu.async_copy(feats_h.at[cur1_v.at[0]], fbuf.at[pl.ds(0, _CH)], fsem)

    def f1loop(k, carry):
        off = (k & 1) * _CH
        pltpu.make_async_copy(feats_h.at[cur1_v.at[k]],
                              fbuf.at[pl.ds(off, _CH)], fsem).wait()

        @pl.when(k < _C1 - 1)
        def _fire():
            off2 = ((k + 1) & 1) * _CH
            pltpu.async_copy(feats_h.at[cur1_v.at[k + 1]],
                             fbuf.at[pl.ds(off2, _CH)], fsem)

        pltpu.sync_copy(fbuf.at[pl.ds(off, _CH)],
                        f1_h.at[pl.ds(g0 + k * _CH, _CH)])
        return carry

    lax.fori_loop(0, _C1, f1loop, 0)

    pltpu.sync_copy(r2_h.at[w], bufa2)

    def gi2(k, carry):
        pltpu.async_copy(cur1_h.at[bufa2.at[k]], bufb2.at[k], gsem)
        return carry

    def di2(k, carry):
        pltpu.make_async_copy(cur1_h.at[bufa2.at[k]], bufb2.at[k],
                              gsem).wait()
        return carry

    lax.fori_loop(0, _C2, gi2, 0)
    lax.fori_loop(0, _C2, di2, 0)
    pltpu.sync_copy(s2_h.at[w], bufa2)

    def ma2(r, carry):
        for seg in range(_CH // 16):
            x = bufb2[r, pl.ds(seg * 16, 16)]
            y = bufa2[r, pl.ds(seg * 16, 16)]
            bufb2[r, pl.ds(seg * 16, 16)] = x * _DEG + y
        return carry

    lax.fori_loop(0, _C2, ma2, 0)

    def gc2(k, carry):
        pltpu.async_copy(adjf_h.at[bufb2.at[k]], cur2_v.at[k], gsem)
        return carry

    def dc2(k, carry):
        pltpu.make_async_copy(adjf_h.at[bufb2.at[k]], cur2_v.at[k],
                              gsem).wait()
        return carry

    lax.fori_loop(0, _C2, gc2, 0)
    lax.fori_loop(0, _C2, dc2, 0)

    ngrp = _CH // _NS2
    pltpu.async_copy(feats_h.at[cur2_v.at[0]], fbuf.at[pl.ds(0, _CH)], fsem)

    def cc(k, carry):
        off = (k & 1) * _CH
        pltpu.make_async_copy(feats_h.at[cur2_v.at[k]],
                              fbuf.at[pl.ds(off, _CH)], fsem).wait()

        @pl.when(k < _C2 - 1)
        def _fire():
            off2 = ((k + 1) & 1) * _CH
            pltpu.async_copy(feats_h.at[cur2_v.at[k + 1]],
                             fbuf.at[pl.ds(off2, _CH)], fsem)

        def grp(g, gc):
            for seg in range(_D // 16):
                acc = fbuf[off + g * _NS2, pl.ds(seg * 16, 16)]
                for j in range(1, _NS2):
                    acc = acc + fbuf[off + g * _NS2 + j, pl.ds(seg * 16, 16)]
                accbuf[g, pl.ds(seg * 16, 16)] = acc
            return gc

        lax.fori_loop(0, ngrp, grp, 0)
        pltpu.sync_copy(accbuf, sum2_h.at[pl.ds(g0 + k * ngrp, ngrp)])
        return carry

    lax.fori_loop(0, _C2, cc, 0)


def _sc_gather(ids, feats, adjf, r1, s1, r2, s2):
    mesh = plsc.VectorSubcoreMesh(core_axis_name="c", subcore_axis_name="s")
    f = pl.kernel(
        _sc_body,
        out_type=(
            jax.ShapeDtypeStruct((_B, _D), jnp.float32),
            jax.ShapeDtypeStruct((_B * _NS1, _D), jnp.float32),
            jax.ShapeDtypeStruct((_B * _NS1, _D), jnp.float32),
            jax.ShapeDtypeStruct((_B * _NS1,), jnp.int32),
        ),
        mesh=mesh,
        scratch_types=[
            pltpu.VMEM((_SEEDS_W,), jnp.int32),
            pltpu.VMEM((_C1, _CH), jnp.int32),
            pltpu.VMEM((_C1, _CH), jnp.int32),
            pltpu.VMEM((_C1, _CH), jnp.int32),
            pltpu.VMEM((_C2, _CH), jnp.int32),
            pltpu.VMEM((_C2, _CH), jnp.int32),
            pltpu.VMEM((_C2, _CH), jnp.int32),
            pltpu.VMEM((2 * _CH, _D), jnp.float32),
            pltpu.VMEM((_CH // _NS2, _D), jnp.float32),
            pltpu.SemaphoreType.DMA,
            pltpu.SemaphoreType.DMA,
        ],
    )
    return f(ids, feats, adjf, r1, s1, r2, s2)


_RB = 1600
_GB = _RB // _NS1
_NSTEP = (_B * _NS1) // _RB


def _tc_body(f0_ref, f1_ref, sum2_ref, ws1_ref, wn1_ref, ws2_ref, wn2_ref,
             fcw_ref, fcb_ref, out_ref, acc_h1, acc_f1):
    i = pl.program_id(0)
    f1 = f1_ref[...]
    s2 = sum2_ref[...]
    ws1 = ws1_ref[...]
    wn1 = wn1_ref[...]
    a = jnp.maximum(jnp.dot(f1, ws1, preferred_element_type=jnp.float32), 0.0)
    b = jnp.maximum(jnp.dot(s2 * (1.0 / _NS2), wn1,
                            preferred_element_type=jnp.float32), 0.0)
    h1 = jnp.concatenate([a, b], axis=1)
    rsel = lax.broadcasted_iota(jnp.int32, (_GB, _RB), 1) // _NS1
    gsel = (rsel == lax.broadcasted_iota(jnp.int32, (_GB, _RB), 0)).astype(jnp.float32)
    acc_h1[pl.ds(i * _GB, _GB), :] = jnp.dot(gsel, h1, preferred_element_type=jnp.float32)
    acc_f1[pl.ds(i * _GB, _GB), :] = jnp.dot(gsel, f1, preferred_element_type=jnp.float32)

    @pl.when(i == _NSTEP - 1)
    def _final():
        inv = 1.0 / _NS1
        f0 = f0_ref[...]
        h0a = jnp.maximum(jnp.dot(f0, ws1, preferred_element_type=jnp.float32), 0.0)
        h0b = jnp.maximum(jnp.dot(acc_f1[...] * inv, wn1,
                                  preferred_element_type=jnp.float32), 0.0)
        h0 = jnp.concatenate([h0a, h0b], axis=1)
        ha = jnp.maximum(jnp.dot(h0, ws2_ref[...], preferred_element_type=jnp.float32), 0.0)
        hb = jnp.maximum(jnp.dot(acc_h1[...] * inv, wn2_ref[...],
                                 preferred_element_type=jnp.float32), 0.0)
        hp = jnp.concatenate([ha, hb], axis=1)
        ss = jnp.sum(hp * hp, axis=1, keepdims=True)
        norm = jnp.maximum(jnp.sqrt(ss), 1e-12)
        out_ref[...] = (jnp.dot(hp / norm, fcw_ref[...],
                                preferred_element_type=jnp.float32) + fcb_ref[...])


def _tc_compute(f0, f1, sum2, ws1, wn1, ws2, wn2, fcw, fcb):
    return pl.pallas_call(
        _tc_body,
        grid=(_NSTEP,),
        in_specs=[
            pl.BlockSpec((_B, _D), lambda i: (0, 0)),
            pl.BlockSpec((_RB, _D), lambda i: (i, 0)),
            pl.BlockSpec((_RB, _D), lambda i: (i, 0)),
            pl.BlockSpec((_D, _HID), lambda i: (0, 0)),
            pl.BlockSpec((_D, _HID), lambda i: (0, 0)),
            pl.BlockSpec((2 * _HID, _HID), lambda i: (0, 0)),
            pl.BlockSpec((2 * _HID, _HID), lambda i: (0, 0)),
            pl.BlockSpec((2 * _HID, 64), lambda i: (0, 0)),
            pl.BlockSpec((1, 64), lambda i: (0, 0)),
        ],
        out_specs=pl.BlockSpec((_B, 64), lambda i: (0, 0)),
        out_shape=jax.ShapeDtypeStruct((_B, 64), jnp.float32),
        scratch_shapes=[
            pltpu.VMEM((_B, 2 * _HID), jnp.float32),
            pltpu.VMEM((_B, _HID), jnp.float32),
        ],
    )(f0, f1, sum2, ws1, wn1, ws2, wn2, fcw, fcb)


def kernel(ids, feats, adj, W_self1, W_neib1, W_self2, W_neib2, fc_W, fc_b):
    ids = ids.astype(jnp.int32)
    adjf = adj.astype(jnp.int32).reshape(-1)
    r1 = jnp.asarray(_R1)
    s1 = jnp.asarray(_SEL1)
    r2 = jnp.asarray(_R2)
    s2 = jnp.asarray(_SEL2)
    f0, f1, sum2, _ = _sc_gather(ids, feats, adjf, r1, s1, r2, s2)
    return _tc_compute(f0, f1, sum2, W_self1, W_neib1, W_self2, W_neib2,
                       fc_W, fc_b.reshape(1, -1))


# speedup vs baseline: 5.5403x; 1.1196x over previous
import jax
import jax.numpy as jnp
import numpy as np
from jax import lax
from jax.experimental import pallas as pl
from jax.experimental.pallas import tpu as pltpu
from jax.experimental.pallas import tpu_sc as plsc

_N_NODES = 100000
_D = 128
_DEG = 32
_B = 1024
_NS1 = 25
_NS2 = 10
_HID = 128
_NW = 32
_SEEDS_W = _B // _NW
_CH = 80
_C1 = (_B * _NS1) // (_NW * _CH)
_C2 = (_B * _NS1 * _NS2) // (_NW * _CH)
_G1_W = _C1 * _CH
_G2_W = _C2 * _CH


def _rotl32(x, d):
    d = np.uint32(d)
    return (x << d) | (x >> np.uint32(32 - d))


def _tf2x32(k1, k2, c1, c2):
    rot = ((13, 15, 26, 6), (17, 29, 16, 24))
    ks = (k1, k2, k1 ^ k2 ^ np.uint32(0x1BD11BDA))
    x0 = (c1 + ks[0]).astype(np.uint32)
    x1 = (c2 + ks[1]).astype(np.uint32)
    for i in range(5):
        for r in rot[i % 2]:
            x0 = (x0 + x1).astype(np.uint32)
            x1 = _rotl32(x1, r)
            x1 = x0 ^ x1
        x0 = (x0 + ks[(i + 1) % 3]).astype(np.uint32)
        x1 = (x1 + ks[(i + 2) % 3] + np.uint32(i + 1)).astype(np.uint32)
    return x0, x1


def _sel_draw(layer, n):
    key = (np.zeros(1, np.uint32), np.full(1, 42, np.uint32))
    key = _tf2x32(key[0], key[1], np.zeros(1, np.uint32),
                  np.full(1, layer, np.uint32))
    b1, b2 = _tf2x32(key[0], key[1], np.zeros(2, np.uint32),
                     np.arange(2, dtype=np.uint32))
    k_hi = (b1[0:1], b2[0:1])
    k_lo = (b1[1:2], b2[1:2])
    idx = np.arange(n, dtype=np.uint64)
    c1 = (idx >> np.uint64(32)).astype(np.uint32)
    c2 = (idx & np.uint64(0xFFFFFFFF)).astype(np.uint32)
    h1, h2 = _tf2x32(k_hi[0], k_hi[1], c1, c2)
    l1, l2 = _tf2x32(k_lo[0], k_lo[1], c1, c2)
    hi = h1 ^ h2
    lo = l1 ^ l2
    return ((hi % np.uint32(_DEG)) * np.uint32(0)
            + (lo % np.uint32(_DEG))).astype(np.int32)


_SEL1 = _sel_draw(0, _B * _NS1).reshape(_NW, _C1, _CH)
_SEL2 = _sel_draw(1, _B * _NS1 * _NS2).reshape(_NW, _C2, _CH)
_R1 = (np.arange(_B * _NS1, dtype=np.int32) // _NS1).reshape(_NW, _C1, _CH)
_R2 = (np.arange(_B * _NS1 * _NS2, dtype=np.int32) // _NS2).reshape(
    _NW, _C2, _CH)
_NSLOT = 4


def _sc_body(ids_h, feats_h, adjf_h, r1_h, s1_h, r2_h, s2_h,
             f0_h, f1_h, sum2_h, cur1_h,
             ids_v, bufa1, bufb1, cur1_v, bufa2, bufb2, cur2_v,
             fbuf, sbuf, accbuf, gsem, fsem, wsem):
    c = lax.axis_index("c")
    s = lax.axis_index("s")
    w = s * 2 + c
    seed0 = w * _SEEDS_W
    g0 = w * _G1_W
    ngrp = _CH // _NS2

    pltpu.sync_copy(ids_h.at[pl.ds(seed0, _SEEDS_W)], ids_v)
    pltpu.async_copy(feats_h.at[ids_v], sbuf, fsem)

    pltpu.sync_copy(r1_h.at[w], bufa1)

    def gi1(k, carry):
        pltpu.async_copy(ids_h.at[bufa1.at[k]], bufb1.at[k], gsem)
        return carry

    def di1(k, carry):
        pltpu.make_async_copy(ids_h.at[bufa1.at[k]], bufb1.at[k], gsem).wait()
        return carry

    lax.fori_loop(0, _C1, gi1, 0)
    lax.fori_loop(0, _C1, di1, 0)
    pltpu.sync_copy(s1_h.at[w], bufa1)

    def ma1(r, carry):
        for seg in range(_CH // 16):
            x = bufb1[r, pl.ds(seg * 16, 16)]
            y = bufa1[r, pl.ds(seg * 16, 16)]
            bufb1[r, pl.ds(seg * 16, 16)] = x * _DEG + y
        return carry

    lax.fori_loop(0, _C1, ma1, 0)

    def gc1(k, carry):
        pltpu.async_copy(adjf_h.at[bufb1.at[k]], cur1_v.at[k], gsem)
        return carry

    def dc1(k, carry):
        pltpu.make_async_copy(adjf_h.at[bufb1.at[k]], cur1_v.at[k],
                              gsem).wait()
        return carry

    lax.fori_loop(0, _C1, gc1, 0)
    lax.fori_loop(0, _C1, dc1, 0)

    def wc1(k, carry):
        pltpu.sync_copy(cur1_v.at[k], cur1_h.at[pl.ds(g0 + k * _CH, _CH)])
        return carry

    lax.fori_loop(0, _C1, wc1, 0)

    pltpu.sync_copy(r2_h.at[w], bufa2)

    def gi2(k, carry):
        pltpu.async_copy(cur1_h.at[bufa2.at[k]], bufb2.at[k], gsem)
        return carry

    lax.fori_loop(0, _C2, gi2, 0)

    pltpu.make_async_copy(feats_h.at[ids_v], sbuf, fsem).wait()
    pltpu.sync_copy(sbuf, f0_h.at[pl.ds(seed0, _SEEDS_W)])

    for p in range(_NSLOT - 1):
        pltpu.async_copy(feats_h.at[cur1_v.at[p]],
                         fbuf.at[pl.ds(p * _CH, _CH)], fsem)

    def f1loop(k, carry):
        off = (k % _NSLOT) * _CH
        pltpu.make_async_copy(feats_h.at[cur1_v.at[k]],
                              fbuf.at[pl.ds(off, _CH)], fsem).wait()

        @pl.when(k >= 1)
        def _drain_w():
            ko = k - 1
            pltpu.make_async_copy(
                fbuf.at[pl.ds((ko % _NSLOT) * _CH, _CH)],
                f1_h.at[pl.ds(g0 + ko * _CH, _CH)], wsem).wait()

        @pl.when(k < _C1 - (_NSLOT - 1))
        def _fire():
            kn = k + _NSLOT - 1
            pltpu.async_copy(feats_h.at[cur1_v.at[kn]],
                             fbuf.at[pl.ds((kn % _NSLOT) * _CH, _CH)], fsem)

        pltpu.async_copy(fbuf.at[pl.ds(off, _CH)],
                         f1_h.at[pl.ds(g0 + k * _CH, _CH)], wsem)
        return carry

    lax.fori_loop(0, _C1, f1loop, 0)
    ko = _C1 - 1
    pltpu.make_async_copy(fbuf.at[pl.ds((ko % _NSLOT) * _CH, _CH)],
                          f1_h.at[pl.ds(g0 + ko * _CH, _CH)], wsem).wait()

    def di2(k, carry):
        pltpu.make_async_copy(cur1_h.at[bufa2.at[k]], bufb2.at[k],
                              gsem).wait()
        return carry

    lax.fori_loop(0, _C2, di2, 0)
    pltpu.sync_copy(s2_h.at[w], bufa2)

    def ma2(r, carry):
        for seg in range(_CH // 16):
            x = bufb2[r, pl.ds(seg * 16, 16)]
            y = bufa2[r, pl.ds(seg * 16, 16)]
            bufb2[r, pl.ds(seg * 16, 16)] = x * _DEG + y
        return carry

    lax.fori_loop(0, _C2, ma2, 0)

    def gc2(k, carry):
        pltpu.async_copy(adjf_h.at[bufb2.at[k]], cur2_v.at[k], gsem)
        return carry

    def dc2(k, carry):
        pltpu.make_async_copy(adjf_h.at[bufb2.at[k]], cur2_v.at[k],
                              gsem).wait()
        return carry

    lax.fori_loop(0, _C2, gc2, 0)
    lax.fori_loop(0, _C2, dc2, 0)

    for p in range(_NSLOT - 1):
        pltpu.async_copy(feats_h.at[cur2_v.at[p]],
                         fbuf.at[pl.ds(p * _CH, _CH)], fsem)

    def cc(k, carry):
        off = (k % _NSLOT) * _CH
        pltpu.make_async_copy(feats_h.at[cur2_v.at[k]],
                              fbuf.at[pl.ds(off, _CH)], fsem).wait()

        @pl.when(k < _C2 - (_NSLOT - 1))
        def _fire():
            kn = k + _NSLOT - 1
            pltpu.async_copy(feats_h.at[cur2_v.at[kn]],
                             fbuf.at[pl.ds((kn % _NSLOT) * _CH, _CH)], fsem)

        @pl.when(k >= 2)
        def _drain_w():
            ko = k - 2
            pltpu.make_async_copy(
                accbuf.at[pl.ds((ko & 1) * ngrp, ngrp)],
                sum2_h.at[pl.ds(g0 + ko * ngrp, ngrp)], wsem).wait()

        aoff = (k & 1) * ngrp

        def grp(g, gc):
            for seg in range(_D // 16):
                acc = fbuf[off + g * _NS2, pl.ds(seg * 16, 16)]
                for j in range(1, _NS2):
                    acc = acc + fbuf[off + g * _NS2 + j, pl.ds(seg * 16, 16)]
                accbuf[aoff + g, pl.ds(seg * 16, 16)] = acc
            return gc

        lax.fori_loop(0, ngrp, grp, 0)
        pltpu.async_copy(accbuf.at[pl.ds(aoff, ngrp)],
                         sum2_h.at[pl.ds(g0 + k * ngrp, ngrp)], wsem)
        return carry

    lax.fori_loop(0, _C2, cc, 0)
    for ko in (_C2 - 2, _C2 - 1):
        pltpu.make_async_copy(accbuf.at[pl.ds((ko & 1) * ngrp, ngrp)],
                              sum2_h.at[pl.ds(g0 + ko * ngrp, ngrp)],
                              wsem).wait()


def _sc_gather(ids, feats, adjf, r1, s1, r2, s2):
    mesh = plsc.VectorSubcoreMesh(core_axis_name="c", subcore_axis_name="s")
    f = pl.kernel(
        _sc_body,
        out_type=(
            jax.ShapeDtypeStruct((_B, _D), jnp.float32),
            jax.ShapeDtypeStruct((_B * _NS1, _D), jnp.float32),
            jax.ShapeDtypeStruct((_B * _NS1, _D), jnp.float32),
            jax.ShapeDtypeStruct((_B * _NS1,), jnp.int32),
        ),
        mesh=mesh,
        scratch_types=[
            pltpu.VMEM((_SEEDS_W,), jnp.int32),
            pltpu.VMEM((_C1, _CH), jnp.int32),
            pltpu.VMEM((_C1, _CH), jnp.int32),
            pltpu.VMEM((_C1, _CH), jnp.int32),
            pltpu.VMEM((_C2, _CH), jnp.int32),
            pltpu.VMEM((_C2, _CH), jnp.int32),
            pltpu.VMEM((_C2, _CH), jnp.int32),
            pltpu.VMEM((_NSLOT * _CH, _D), jnp.float32),
            pltpu.VMEM((_SEEDS_W, _D), jnp.float32),
            pltpu.VMEM((2 * (_CH // _NS2), _D), jnp.float32),
            pltpu.SemaphoreType.DMA,
            pltpu.SemaphoreType.DMA,
            pltpu.SemaphoreType.DMA,
        ],
    )
    return f(ids, feats, adjf, r1, s1, r2, s2)


_RB = 1600
_GB = _RB // _NS1
_NSTEP = (_B * _NS1) // _RB


def _tc_body(f0_ref, f1_ref, sum2_ref, ws1_ref, wn1_ref, ws2_ref, wn2_ref,
             fcw_ref, fcb_ref, out_ref, acc_h1, acc_f1):
    i = pl.program_id(0)
    f1 = f1_ref[...]
    s2 = sum2_ref[...]
    ws1 = ws1_ref[...]
    wn1 = wn1_ref[...]
    a = jnp.maximum(jnp.dot(f1, ws1, preferred_element_type=jnp.float32), 0.0)
    b = jnp.maximum(jnp.dot(s2 * (1.0 / _NS2), wn1,
                            preferred_element_type=jnp.float32), 0.0)
    h1 = jnp.concatenate([a, b], axis=1)
    rsel = lax.broadcasted_iota(jnp.int32, (_GB, _RB), 1) // _NS1
    gsel = (rsel == lax.broadcasted_iota(jnp.int32, (_GB, _RB), 0)).astype(jnp.float32)
    acc_h1[pl.ds(i * _GB, _GB), :] = jnp.dot(gsel, h1, preferred_element_type=jnp.float32)
    acc_f1[pl.ds(i * _GB, _GB), :] = jnp.dot(gsel, f1, preferred_element_type=jnp.float32)

    @pl.when(i == _NSTEP - 1)
    def _final():
        inv = 1.0 / _NS1
        f0 = f0_ref[...]
        h0a = jnp.maximum(jnp.dot(f0, ws1, preferred_element_type=jnp.float32), 0.0)
        h0b = jnp.maximum(jnp.dot(acc_f1[...] * inv, wn1,
                                  preferred_element_type=jnp.float32), 0.0)
        h0 = jnp.concatenate([h0a, h0b], axis=1)
        ha = jnp.maximum(jnp.dot(h0, ws2_ref[...], preferred_element_type=jnp.float32), 0.0)
        hb = jnp.maximum(jnp.dot(acc_h1[...] * inv, wn2_ref[...],
                                 preferred_element_type=jnp.float32), 0.0)
        hp = jnp.concatenate([ha, hb], axis=1)
        ss = jnp.sum(hp * hp, axis=1, keepdims=True)
        norm = jnp.maximum(jnp.sqrt(ss), 1e-12)
        out_ref[...] = (jnp.dot(hp / norm, fcw_ref[...],
                                preferred_element_type=jnp.float32) + fcb_ref[...])


def _tc_compute(f0, f1, sum2, ws1, wn1, ws2, wn2, fcw, fcb):
    return pl.pallas_call(
        _tc_body,
        grid=(_NSTEP,),
        in_specs=[
            pl.BlockSpec((_B, _D), lambda i: (0, 0)),
            pl.BlockSpec((_RB, _D), lambda i: (i, 0)),
            pl.BlockSpec((_RB, _D), lambda i: (i, 0)),
            pl.BlockSpec((_D, _HID), lambda i: (0, 0)),
            pl.BlockSpec((_D, _HID), lambda i: (0, 0)),
            pl.BlockSpec((2 * _HID, _HID), lambda i: (0, 0)),
            pl.BlockSpec((2 * _HID, _HID), lambda i: (0, 0)),
            pl.BlockSpec((2 * _HID, 64), lambda i: (0, 0)),
            pl.BlockSpec((1, 64), lambda i: (0, 0)),
        ],
        out_specs=pl.BlockSpec((_B, 64), lambda i: (0, 0)),
        out_shape=jax.ShapeDtypeStruct((_B, 64), jnp.float32),
        scratch_shapes=[
            pltpu.VMEM((_B, 2 * _HID), jnp.float32),
            pltpu.VMEM((_B, _HID), jnp.float32),
        ],
    )(f0, f1, sum2, ws1, wn1, ws2, wn2, fcw, fcb)


def kernel(ids, feats, adj, W_self1, W_neib1, W_self2, W_neib2, fc_W, fc_b):
    ids = ids.astype(jnp.int32)
    adjf = adj.astype(jnp.int32).reshape(-1)
    r1 = jnp.asarray(_R1)
    s1 = jnp.asarray(_SEL1)
    r2 = jnp.asarray(_R2)
    s2 = jnp.asarray(_SEL2)
    f0, f1, sum2, _ = _sc_gather(ids, feats, adjf, r1, s1, r2, s2)
    return _tc_compute(f0, f1, sum2, W_self1, W_neib1, W_self2, W_neib2,
                       fc_W, fc_b.reshape(1, -1))


# speedup vs baseline: 6.5309x; 1.1788x over previous
import jax
import jax.numpy as jnp
import numpy as np
from jax import lax
from jax.experimental import pallas as pl
from jax.experimental.pallas import tpu as pltpu
from jax.experimental.pallas import tpu_sc as plsc

_N_NODES = 100000
_D = 128
_DEG = 32
_B = 1024
_NS1 = 25
_NS2 = 10
_HID = 128
_NW = 32
_SEEDS_W = _B // _NW
_CH = 80
_C1 = (_B * _NS1) // (_NW * _CH)
_C2 = (_B * _NS1 * _NS2) // (_NW * _CH)
_G1_W = _C1 * _CH
_G2_W = _C2 * _CH


def _rotl32(x, d):
    d = np.uint32(d)
    return (x << d) | (x >> np.uint32(32 - d))


def _tf2x32(k1, k2, c1, c2):
    rot = ((13, 15, 26, 6), (17, 29, 16, 24))
    ks = (k1, k2, k1 ^ k2 ^ np.uint32(0x1BD11BDA))
    x0 = (c1 + ks[0]).astype(np.uint32)
    x1 = (c2 + ks[1]).astype(np.uint32)
    for i in range(5):
        for r in rot[i % 2]:
            x0 = (x0 + x1).astype(np.uint32)
            x1 = _rotl32(x1, r)
            x1 = x0 ^ x1
        x0 = (x0 + ks[(i + 1) % 3]).astype(np.uint32)
        x1 = (x1 + ks[(i + 2) % 3] + np.uint32(i + 1)).astype(np.uint32)
    return x0, x1


def _sel_draw(layer, n):
    key = (np.zeros(1, np.uint32), np.full(1, 42, np.uint32))
    key = _tf2x32(key[0], key[1], np.zeros(1, np.uint32),
                  np.full(1, layer, np.uint32))
    b1, b2 = _tf2x32(key[0], key[1], np.zeros(2, np.uint32),
                     np.arange(2, dtype=np.uint32))
    k_hi = (b1[0:1], b2[0:1])
    k_lo = (b1[1:2], b2[1:2])
    idx = np.arange(n, dtype=np.uint64)
    c1 = (idx >> np.uint64(32)).astype(np.uint32)
    c2 = (idx & np.uint64(0xFFFFFFFF)).astype(np.uint32)
    h1, h2 = _tf2x32(k_hi[0], k_hi[1], c1, c2)
    l1, l2 = _tf2x32(k_lo[0], k_lo[1], c1, c2)
    hi = h1 ^ h2
    lo = l1 ^ l2
    return ((hi % np.uint32(_DEG)) * np.uint32(0)
            + (lo % np.uint32(_DEG))).astype(np.int32)


_SEL1 = _sel_draw(0, _B * _NS1).reshape(_NW, _C1, _CH)
_R1 = (np.arange(_B * _NS1, dtype=np.int32) // _NS1).reshape(_NW, _C1, _CH)


def _grp_transpose(t):
    return (t.reshape(_NW, _C2, _CH // _NS2, _NS2)
            .transpose(0, 1, 3, 2).reshape(_NW, _C2, _CH).copy())


_SEL2 = _grp_transpose(_sel_draw(1, _B * _NS1 * _NS2).reshape(_NW, _C2, _CH))
_R2 = _grp_transpose(
    (np.arange(_B * _NS1 * _NS2, dtype=np.int32) // _NS2).reshape(
        _NW, _C2, _CH))
_NSLOT = 4


def _sc_body(ids_h, feats_h, adjf_h, r1_h, s1_h, r2_h, s2_h,
             f0_h, f1_h, sum2_h, cur1_h,
             ids_v, bufa1, bufb1, cur1_v, bufa2, bufb2, cur2_v,
             fbuf, sbuf, accbuf, gsem, fsem, wsem):
    c = lax.axis_index("c")
    s = lax.axis_index("s")
    w = s * 2 + c
    seed0 = w * _SEEDS_W
    g0 = w * _G1_W
    ngrp = _CH // _NS2

    pltpu.sync_copy(ids_h.at[pl.ds(seed0, _SEEDS_W)], ids_v)
    pltpu.async_copy(feats_h.at[ids_v], sbuf, fsem)

    pltpu.sync_copy(r1_h.at[w], bufa1)

    def gi1(k, carry):
        pltpu.async_copy(ids_h.at[bufa1.at[k]], bufb1.at[k], gsem)
        return carry

    def di1(k, carry):
        pltpu.make_async_copy(ids_h.at[bufa1.at[k]], bufb1.at[k], gsem).wait()
        return carry

    lax.fori_loop(0, _C1, gi1, 0)
    lax.fori_loop(0, _C1, di1, 0)
    pltpu.sync_copy(s1_h.at[w], bufa1)

    def ma1(r, carry):
        for seg in range(_CH // 16):
            x = bufb1[r, pl.ds(seg * 16, 16)]
            y = bufa1[r, pl.ds(seg * 16, 16)]
            bufb1[r, pl.ds(seg * 16, 16)] = x * _DEG + y
        return carry

    lax.fori_loop(0, _C1, ma1, 0)

    def gc1(k, carry):
        pltpu.async_copy(adjf_h.at[bufb1.at[k]], cur1_v.at[k], gsem)
        return carry

    def dc1(k, carry):
        pltpu.make_async_copy(adjf_h.at[bufb1.at[k]], cur1_v.at[k],
                              gsem).wait()
        return carry

    lax.fori_loop(0, _C1, gc1, 0)
    lax.fori_loop(0, _C1, dc1, 0)

    def wc1(k, carry):
        pltpu.sync_copy(cur1_v.at[k], cur1_h.at[pl.ds(g0 + k * _CH, _CH)])
        return carry

    lax.fori_loop(0, _C1, wc1, 0)

    pltpu.sync_copy(r2_h.at[w], bufa2)

    def gi2(k, carry):
        pltpu.async_copy(cur1_h.at[bufa2.at[k]], bufb2.at[k], gsem)
        return carry

    lax.fori_loop(0, _C2, gi2, 0)

    pltpu.make_async_copy(feats_h.at[ids_v], sbuf, fsem).wait()
    pltpu.sync_copy(sbuf, f0_h.at[pl.ds(seed0, _SEEDS_W)])

    for p in range(_NSLOT - 1):
        pltpu.async_copy(feats_h.at[cur1_v.at[p]],
                         fbuf.at[pl.ds(p * _CH, _CH)], fsem)

    def f1loop(k, carry):
        off = (k % _NSLOT) * _CH
        pltpu.make_async_copy(feats_h.at[cur1_v.at[k]],
                              fbuf.at[pl.ds(off, _CH)], fsem).wait()

        @pl.when(k >= 1)
        def _drain_w():
            ko = k - 1
            pltpu.make_async_copy(
                fbuf.at[pl.ds((ko % _NSLOT) * _CH, _CH)],
                f1_h.at[pl.ds(g0 + ko * _CH, _CH)], wsem).wait()

        @pl.when(k < _C1 - (_NSLOT - 1))
        def _fire():
            kn = k + _NSLOT - 1
            pltpu.async_copy(feats_h.at[cur1_v.at[kn]],
                             fbuf.at[pl.ds((kn % _NSLOT) * _CH, _CH)], fsem)

        pltpu.async_copy(fbuf.at[pl.ds(off, _CH)],
                         f1_h.at[pl.ds(g0 + k * _CH, _CH)], wsem)
        return carry

    lax.fori_loop(0, _C1, f1loop, 0)
    ko = _C1 - 1
    pltpu.make_async_copy(fbuf.at[pl.ds((ko % _NSLOT) * _CH, _CH)],
                          f1_h.at[pl.ds(g0 + ko * _CH, _CH)], wsem).wait()

    def di2(k, carry):
        pltpu.make_async_copy(cur1_h.at[bufa2.at[k]], bufb2.at[k],
                              gsem).wait()
        return carry

    lax.fori_loop(0, _C2, di2, 0)
    pltpu.sync_copy(s2_h.at[w], bufa2)

    def ma2(r, carry):
        for seg in range(_CH // 16):
            x = bufb2[r, pl.ds(seg * 16, 16)]
            y = bufa2[r, pl.ds(seg * 16, 16)]
            bufb2[r, pl.ds(seg * 16, 16)] = x * _DEG + y
        return carry

    lax.fori_loop(0, _C2, ma2, 0)

    def gc2(k, carry):
        pltpu.async_copy(adjf_h.at[bufb2.at[k]], cur2_v.at[k], gsem)
        return carry

    def dc2(k, carry):
        pltpu.make_async_copy(adjf_h.at[bufb2.at[k]], cur2_v.at[k],
                              gsem).wait()
        return carry

    lax.fori_loop(0, _C2, gc2, 0)
    lax.fori_loop(0, _C2, dc2, 0)

    def l2fire(k, slot):
        for r in range(ngrp):
            for seg in range(_D // 16):
                accbuf[slot * ngrp + r, pl.ds(seg * 16, 16)] = jnp.zeros(
                    (16,), jnp.float32)
        for j in range(_NS2):
            pltpu.async_copy(
                feats_h.at[cur2_v.at[k, pl.ds(j * ngrp, ngrp)]],
                accbuf.at[pl.ds(slot * ngrp, ngrp)], fsem,
                add=True)

    def l2wait(k, slot):
        for j in range(_NS2):
            pltpu.make_async_copy(
                feats_h.at[cur2_v.at[k, pl.ds(j * ngrp, ngrp)]],
                accbuf.at[pl.ds(slot * ngrp, ngrp)], fsem).wait()

    for p in range(_NSLOT - 1):
        l2fire(p, p)

    def cc(k, carry):
        slot = k % _NSLOT
        l2wait(k, slot)

        @pl.when(k >= 1)
        def _drain_w():
            ko = k - 1
            pltpu.make_async_copy(
                accbuf.at[pl.ds((ko % _NSLOT) * ngrp, ngrp)],
                sum2_h.at[pl.ds(g0 + ko * ngrp, ngrp)], wsem).wait()

        @pl.when(k < _C2 - (_NSLOT - 1))
        def _fire():
            kn = k + _NSLOT - 1
            l2fire(kn, kn % _NSLOT)

        pltpu.async_copy(accbuf.at[pl.ds(slot * ngrp, ngrp)],
                         sum2_h.at[pl.ds(g0 + k * ngrp, ngrp)], wsem)
        return carry

    lax.fori_loop(0, _C2, cc, 0)
    ko = _C2 - 1
    pltpu.make_async_copy(accbuf.at[pl.ds((ko % _NSLOT) * ngrp, ngrp)],
                          sum2_h.at[pl.ds(g0 + ko * ngrp, ngrp)],
                          wsem).wait()


def _sc_gather(ids, feats, adjf, r1, s1, r2, s2):
    mesh = plsc.VectorSubcoreMesh(core_axis_name="c", subcore_axis_name="s")
    f = pl.kernel(
        _sc_body,
        out_type=(
            jax.ShapeDtypeStruct((_B, _D), jnp.float32),
            jax.ShapeDtypeStruct((_B * _NS1, _D), jnp.float32),
            jax.ShapeDtypeStruct((_B * _NS1, _D), jnp.float32),
            jax.ShapeDtypeStruct((_B * _NS1,), jnp.int32),
        ),
        mesh=mesh,
        scratch_types=[
            pltpu.VMEM((_SEEDS_W,), jnp.int32),
            pltpu.VMEM((_C1, _CH), jnp.int32),
            pltpu.VMEM((_C1, _CH), jnp.int32),
            pltpu.VMEM((_C1, _CH), jnp.int32),
            pltpu.VMEM((_C2, _CH), jnp.int32),
            pltpu.VMEM((_C2, _CH), jnp.int32),
            pltpu.VMEM((_C2, _CH), jnp.int32),
            pltpu.VMEM((_NSLOT * _CH, _D), jnp.float32),
            pltpu.VMEM((_SEEDS_W, _D), jnp.float32),
            pltpu.VMEM((_NSLOT * (_CH // _NS2), _D), jnp.float32),
            pltpu.SemaphoreType.DMA,
            pltpu.SemaphoreType.DMA,
            pltpu.SemaphoreType.DMA,
        ],
    )
    return f(ids, feats, adjf, r1, s1, r2, s2)


_RB = 1600
_GB = _RB // _NS1
_NSTEP = (_B * _NS1) // _RB


def _tc_body(f0_ref, f1_ref, sum2_ref, ws1_ref, wn1_ref, ws2_ref, wn2_ref,
             fcw_ref, fcb_ref, out_ref, acc_h1, acc_f1):
    i = pl.program_id(0)
    f1 = f1_ref[...]
    s2 = sum2_ref[...]
    ws1 = ws1_ref[...]
    wn1 = wn1_ref[...]
    a = jnp.maximum(jnp.dot(f1, ws1, preferred_element_type=jnp.float32), 0.0)
    b = jnp.maximum(jnp.dot(s2 * (1.0 / _NS2), wn1,
                            preferred_element_type=jnp.float32), 0.0)
    h1 = jnp.concatenate([a, b], axis=1)
    rsel = lax.broadcasted_iota(jnp.int32, (_GB, _RB), 1) // _NS1
    gsel = (rsel == lax.broadcasted_iota(jnp.int32, (_GB, _RB), 0)).astype(jnp.float32)
    acc_h1[pl.ds(i * _GB, _GB), :] = jnp.dot(gsel, h1, preferred_element_type=jnp.float32)
    acc_f1[pl.ds(i * _GB, _GB), :] = jnp.dot(gsel, f1, preferred_element_type=jnp.float32)

    @pl.when(i == _NSTEP - 1)
    def _final():
        inv = 1.0 / _NS1
        f0 = f0_ref[...]
        h0a = jnp.maximum(jnp.dot(f0, ws1, preferred_element_type=jnp.float32), 0.0)
        h0b = jnp.maximum(jnp.dot(acc_f1[...] * inv, wn1,
                                  preferred_element_type=jnp.float32), 0.0)
        h0 = jnp.concatenate([h0a, h0b], axis=1)
        ha = jnp.maximum(jnp.dot(h0, ws2_ref[...], preferred_element_type=jnp.float32), 0.0)
        hb = jnp.maximum(jnp.dot(acc_h1[...] * inv, wn2_ref[...],
                                 preferred_element_type=jnp.float32), 0.0)
        hp = jnp.concatenate([ha, hb], axis=1)
        ss = jnp.sum(hp * hp, axis=1, keepdims=True)
        norm = jnp.maximum(jnp.sqrt(ss), 1e-12)
        out_ref[...] = (jnp.dot(hp / norm, fcw_ref[...],
                                preferred_element_type=jnp.float32) + fcb_ref[...])


def _tc_compute(f0, f1, sum2, ws1, wn1, ws2, wn2, fcw, fcb):
    return pl.pallas_call(
        _tc_body,
        grid=(_NSTEP,),
        in_specs=[
            pl.BlockSpec((_B, _D), lambda i: (0, 0)),
            pl.BlockSpec((_RB, _D), lambda i: (i, 0)),
            pl.BlockSpec((_RB, _D), lambda i: (i, 0)),
            pl.BlockSpec((_D, _HID), lambda i: (0, 0)),
            pl.BlockSpec((_D, _HID), lambda i: (0, 0)),
            pl.BlockSpec((2 * _HID, _HID), lambda i: (0, 0)),
            pl.BlockSpec((2 * _HID, _HID), lambda i: (0, 0)),
            pl.BlockSpec((2 * _HID, 64), lambda i: (0, 0)),
            pl.BlockSpec((1, 64), lambda i: (0, 0)),
        ],
        out_specs=pl.BlockSpec((_B, 64), lambda i: (0, 0)),
        out_shape=jax.ShapeDtypeStruct((_B, 64), jnp.float32),
        scratch_shapes=[
            pltpu.VMEM((_B, 2 * _HID), jnp.float32),
            pltpu.VMEM((_B, _HID), jnp.float32),
        ],
    )(f0, f1, sum2, ws1, wn1, ws2, wn2, fcw, fcb)


def kernel(ids, feats, adj, W_self1, W_neib1, W_self2, W_neib2, fc_W, fc_b):
    ids = ids.astype(jnp.int32)
    adjf = adj.astype(jnp.int32).reshape(-1)
    r1 = jnp.asarray(_R1)
    s1 = jnp.asarray(_SEL1)
    r2 = jnp.asarray(_R2)
    s2 = jnp.asarray(_SEL2)
    f0, f1, sum2, _ = _sc_gather(ids, feats, adjf, r1, s1, r2, s2)
    return _tc_compute(f0, f1, sum2, W_self1, W_neib1, W_self2, W_neib2,
                       fc_W, fc_b.reshape(1, -1))


# speedup vs baseline: 6.6164x; 1.0131x over previous
import jax
import jax.numpy as jnp
import numpy as np
from jax import lax
from jax.experimental import pallas as pl
from jax.experimental.pallas import tpu as pltpu
from jax.experimental.pallas import tpu_sc as plsc

_N_NODES = 100000
_D = 128
_DEG = 32
_B = 1024
_NS1 = 25
_NS2 = 10
_HID = 128
_NW = 32
_SEEDS_W = _B // _NW
_CH = 80
_C1 = (_B * _NS1) // (_NW * _CH)
_C2 = (_B * _NS1 * _NS2) // (_NW * _CH)
_G1_W = _C1 * _CH
_G2_W = _C2 * _CH


def _rotl32(x, d):
    d = np.uint32(d)
    return (x << d) | (x >> np.uint32(32 - d))


def _tf2x32(k1, k2, c1, c2):
    rot = ((13, 15, 26, 6), (17, 29, 16, 24))
    ks = (k1, k2, k1 ^ k2 ^ np.uint32(0x1BD11BDA))
    x0 = (c1 + ks[0]).astype(np.uint32)
    x1 = (c2 + ks[1]).astype(np.uint32)
    for i in range(5):
        for r in rot[i % 2]:
            x0 = (x0 + x1).astype(np.uint32)
            x1 = _rotl32(x1, r)
            x1 = x0 ^ x1
        x0 = (x0 + ks[(i + 1) % 3]).astype(np.uint32)
        x1 = (x1 + ks[(i + 2) % 3] + np.uint32(i + 1)).astype(np.uint32)
    return x0, x1


def _sel_draw(layer, n):
    key = (np.zeros(1, np.uint32), np.full(1, 42, np.uint32))
    key = _tf2x32(key[0], key[1], np.zeros(1, np.uint32),
                  np.full(1, layer, np.uint32))
    b1, b2 = _tf2x32(key[0], key[1], np.zeros(2, np.uint32),
                     np.arange(2, dtype=np.uint32))
    k_hi = (b1[0:1], b2[0:1])
    k_lo = (b1[1:2], b2[1:2])
    idx = np.arange(n, dtype=np.uint64)
    c1 = (idx >> np.uint64(32)).astype(np.uint32)
    c2 = (idx & np.uint64(0xFFFFFFFF)).astype(np.uint32)
    h1, h2 = _tf2x32(k_hi[0], k_hi[1], c1, c2)
    l1, l2 = _tf2x32(k_lo[0], k_lo[1], c1, c2)
    hi = h1 ^ h2
    lo = l1 ^ l2
    return ((hi % np.uint32(_DEG)) * np.uint32(0)
            + (lo % np.uint32(_DEG))).astype(np.int32)


_SEL1 = _sel_draw(0, _B * _NS1).reshape(_NW, _C1, _CH)
_R1 = (np.arange(_B * _NS1, dtype=np.int32) // _NS1).reshape(_NW, _C1, _CH)


def _grp_transpose(t):
    return (t.reshape(_NW, _C2, _CH // _NS2, _NS2)
            .transpose(0, 1, 3, 2).reshape(_NW, _C2, _CH).copy())


_SEL2 = _grp_transpose(_sel_draw(1, _B * _NS1 * _NS2).reshape(_NW, _C2, _CH))
_R2 = _grp_transpose(
    (np.arange(_B * _NS1 * _NS2, dtype=np.int32) // _NS2).reshape(
        _NW, _C2, _CH))
_NSLOT = 6


def _sc_body(ids_h, feats_h, adjf_h, r1_h, s1_h, r2_h, s2_h,
             f0_h, f1_h, sum2_h, cur1_h,
             ids_v, bufa1, bufb1, cur1_v, bufa2, bufb2, cur2_v,
             fbuf, sbuf, accbuf, gsem, fsem, wsem):
    c = lax.axis_index("c")
    s = lax.axis_index("s")
    w = s * 2 + c
    seed0 = w * _SEEDS_W
    g0 = w * _G1_W
    ngrp = _CH // _NS2

    pltpu.sync_copy(ids_h.at[pl.ds(seed0, _SEEDS_W)], ids_v)
    pltpu.async_copy(feats_h.at[ids_v], sbuf, fsem)

    pltpu.sync_copy(r1_h.at[w], bufa1)

    def gi1(k, carry):
        pltpu.async_copy(ids_h.at[bufa1.at[k]], bufb1.at[k], gsem)
        return carry

    def di1(k, carry):
        pltpu.make_async_copy(ids_h.at[bufa1.at[k]], bufb1.at[k], gsem).wait()
        return carry

    lax.fori_loop(0, _C1, gi1, 0)
    lax.fori_loop(0, _C1, di1, 0)
    pltpu.sync_copy(s1_h.at[w], bufa1)

    def ma1(r, carry):
        for seg in range(_CH // 16):
            x = bufb1[r, pl.ds(seg * 16, 16)]
            y = bufa1[r, pl.ds(seg * 16, 16)]
            bufb1[r, pl.ds(seg * 16, 16)] = x * _DEG + y
        return carry

    lax.fori_loop(0, _C1, ma1, 0)

    def gc1(k, carry):
        pltpu.async_copy(adjf_h.at[bufb1.at[k]], cur1_v.at[k], gsem)
        return carry

    def dc1(k, carry):
        pltpu.make_async_copy(adjf_h.at[bufb1.at[k]], cur1_v.at[k],
                              gsem).wait()
        return carry

    lax.fori_loop(0, _C1, gc1, 0)
    lax.fori_loop(0, _C1, dc1, 0)

    def wc1(k, carry):
        pltpu.sync_copy(cur1_v.at[k], cur1_h.at[pl.ds(g0 + k * _CH, _CH)])
        return carry

    lax.fori_loop(0, _C1, wc1, 0)

    pltpu.sync_copy(r2_h.at[w], bufa2)

    def gi2(k, carry):
        pltpu.async_copy(cur1_h.at[bufa2.at[k]], bufb2.at[k], gsem)
        return carry

    lax.fori_loop(0, _C2, gi2, 0)

    pltpu.make_async_copy(feats_h.at[ids_v], sbuf, fsem).wait()
    pltpu.sync_copy(sbuf, f0_h.at[pl.ds(seed0, _SEEDS_W)])

    for p in range(_NSLOT - 1):
        pltpu.async_copy(feats_h.at[cur1_v.at[p]],
                         fbuf.at[pl.ds(p * _CH, _CH)], fsem)

    def f1loop(k, carry):
        off = (k % _NSLOT) * _CH
        pltpu.make_async_copy(feats_h.at[cur1_v.at[k]],
                              fbuf.at[pl.ds(off, _CH)], fsem).wait()

        @pl.when(k >= 1)
        def _drain_w():
            ko = k - 1
            pltpu.make_async_copy(
                fbuf.at[pl.ds((ko % _NSLOT) * _CH, _CH)],
                f1_h.at[pl.ds(g0 + ko * _CH, _CH)], wsem).wait()

        @pl.when(k < _C1 - (_NSLOT - 1))
        def _fire():
            kn = k + _NSLOT - 1
            pltpu.async_copy(feats_h.at[cur1_v.at[kn]],
                             fbuf.at[pl.ds((kn % _NSLOT) * _CH, _CH)], fsem)

        pltpu.async_copy(fbuf.at[pl.ds(off, _CH)],
                         f1_h.at[pl.ds(g0 + k * _CH, _CH)], wsem)
        return carry

    lax.fori_loop(0, _C1, f1loop, 0)
    ko = _C1 - 1
    pltpu.make_async_copy(fbuf.at[pl.ds((ko % _NSLOT) * _CH, _CH)],
                          f1_h.at[pl.ds(g0 + ko * _CH, _CH)], wsem).wait()

    def di2(k, carry):
        pltpu.make_async_copy(cur1_h.at[bufa2.at[k]], bufb2.at[k],
                              gsem).wait()
        return carry

    lax.fori_loop(0, _C2, di2, 0)
    pltpu.sync_copy(s2_h.at[w], bufa2)

    def ma2(r, carry):
        for seg in range(_CH // 16):
            x = bufb2[r, pl.ds(seg * 16, 16)]
            y = bufa2[r, pl.ds(seg * 16, 16)]
            bufb2[r, pl.ds(seg * 16, 16)] = x * _DEG + y
        return carry

    lax.fori_loop(0, _C2, ma2, 0)

    def gc2(k, carry):
        pltpu.async_copy(adjf_h.at[bufb2.at[k]], cur2_v.at[k], gsem)
        return carry

    def dc2(k, carry):
        pltpu.make_async_copy(adjf_h.at[bufb2.at[k]], cur2_v.at[k],
                              gsem).wait()
        return carry

    lax.fori_loop(0, _C2, gc2, 0)
    lax.fori_loop(0, _C2, dc2, 0)

    def l2fire(k, slot):
        for r in range(ngrp):
            for seg in range(_D // 16):
                accbuf[slot * ngrp + r, pl.ds(seg * 16, 16)] = jnp.zeros(
                    (16,), jnp.float32)
        for j in range(_NS2):
            pltpu.async_copy(
                feats_h.at[cur2_v.at[k, pl.ds(j * ngrp, ngrp)]],
                accbuf.at[pl.ds(slot * ngrp, ngrp)], fsem,
                add=True)

    def l2wait(k, slot):
        for j in range(_NS2):
            pltpu.make_async_copy(
                feats_h.at[cur2_v.at[k, pl.ds(j * ngrp, ngrp)]],
                accbuf.at[pl.ds(slot * ngrp, ngrp)], fsem).wait()

    for p in range(_NSLOT - 1):
        l2fire(p, p)

    def cc(k, carry):
        slot = k % _NSLOT
        l2wait(k, slot)

        @pl.when(k >= 1)
        def _drain_w():
            ko = k - 1
            pltpu.make_async_copy(
                accbuf.at[pl.ds((ko % _NSLOT) * ngrp, ngrp)],
                sum2_h.at[pl.ds(g0 + ko * ngrp, ngrp)], wsem).wait()

        @pl.when(k < _C2 - (_NSLOT - 1))
        def _fire():
            kn = k + _NSLOT - 1
            l2fire(kn, kn % _NSLOT)

        pltpu.async_copy(accbuf.at[pl.ds(slot * ngrp, ngrp)],
                         sum2_h.at[pl.ds(g0 + k * ngrp, ngrp)], wsem)
        return carry

    lax.fori_loop(0, _C2, cc, 0)
    ko = _C2 - 1
    pltpu.make_async_copy(accbuf.at[pl.ds((ko % _NSLOT) * ngrp, ngrp)],
                          sum2_h.at[pl.ds(g0 + ko * ngrp, ngrp)],
                          wsem).wait()


def _sc_gather(ids, feats, adjf, r1, s1, r2, s2):
    mesh = plsc.VectorSubcoreMesh(core_axis_name="c", subcore_axis_name="s")
    f = pl.kernel(
        _sc_body,
        out_type=(
            jax.ShapeDtypeStruct((_B, _D), jnp.float32),
            jax.ShapeDtypeStruct((_B * _NS1, _D), jnp.float32),
            jax.ShapeDtypeStruct((_B * _NS1, _D), jnp.float32),
            jax.ShapeDtypeStruct((_B * _NS1,), jnp.int32),
        ),
        mesh=mesh,
        scratch_types=[
            pltpu.VMEM((_SEEDS_W,), jnp.int32),
            pltpu.VMEM((_C1, _CH), jnp.int32),
            pltpu.VMEM((_C1, _CH), jnp.int32),
            pltpu.VMEM((_C1, _CH), jnp.int32),
            pltpu.VMEM((_C2, _CH), jnp.int32),
            pltpu.VMEM((_C2, _CH), jnp.int32),
            pltpu.VMEM((_C2, _CH), jnp.int32),
            pltpu.VMEM((_NSLOT * _CH, _D), jnp.float32),
            pltpu.VMEM((_SEEDS_W, _D), jnp.float32),
            pltpu.VMEM((_NSLOT * (_CH // _NS2), _D), jnp.float32),
            pltpu.SemaphoreType.DMA,
            pltpu.SemaphoreType.DMA,
            pltpu.SemaphoreType.DMA,
        ],
    )
    return f(ids, feats, adjf, r1, s1, r2, s2)


_RB = 1600
_GB = _RB // _NS1
_NSTEP = (_B * _NS1) // _RB


def _tc_body(f0_ref, f1_ref, sum2_ref, ws1_ref, wn1_ref, ws2_ref, wn2_ref,
             fcw_ref, fcb_ref, out_ref, acc_h1, acc_f1):
    i = pl.program_id(0)
    f1 = f1_ref[...]
    s2 = sum2_ref[...]
    ws1 = ws1_ref[...]
    wn1 = wn1_ref[...]
    a = jnp.maximum(jnp.dot(f1, ws1, preferred_element_type=jnp.float32), 0.0)
    b = jnp.maximum(jnp.dot(s2 * (1.0 / _NS2), wn1,
                            preferred_element_type=jnp.float32), 0.0)
    h1 = jnp.concatenate([a, b], axis=1)
    rsel = lax.broadcasted_iota(jnp.int32, (_GB, _RB), 1) // _NS1
    gsel = (rsel == lax.broadcasted_iota(jnp.int32, (_GB, _RB), 0)).astype(jnp.float32)
    acc_h1[pl.ds(i * _GB, _GB), :] = jnp.dot(gsel, h1, preferred_element_type=jnp.float32)
    acc_f1[pl.ds(i * _GB, _GB), :] = jnp.dot(gsel, f1, preferred_element_type=jnp.float32)

    @pl.when(i == _NSTEP - 1)
    def _final():
        inv = 1.0 / _NS1
        f0 = f0_ref[...]
        h0a = jnp.maximum(jnp.dot(f0, ws1, preferred_element_type=jnp.float32), 0.0)
        h0b = jnp.maximum(jnp.dot(acc_f1[...] * inv, wn1,
                                  preferred_element_type=jnp.float32), 0.0)
        h0 = jnp.concatenate([h0a, h0b], axis=1)
        ha = jnp.maximum(jnp.dot(h0, ws2_ref[...], preferred_element_type=jnp.float32), 0.0)
        hb = jnp.maximum(jnp.dot(acc_h1[...] * inv, wn2_ref[...],
                                 preferred_element_type=jnp.float32), 0.0)
        hp = jnp.concatenate([ha, hb], axis=1)
        ss = jnp.sum(hp * hp, axis=1, keepdims=True)
        norm = jnp.maximum(jnp.sqrt(ss), 1e-12)
        out_ref[...] = (jnp.dot(hp / norm, fcw_ref[...],
                                preferred_element_type=jnp.float32) + fcb_ref[...])


def _tc_compute(f0, f1, sum2, ws1, wn1, ws2, wn2, fcw, fcb):
    return pl.pallas_call(
        _tc_body,
        grid=(_NSTEP,),
        in_specs=[
            pl.BlockSpec((_B, _D), lambda i: (0, 0)),
            pl.BlockSpec((_RB, _D), lambda i: (i, 0)),
            pl.BlockSpec((_RB, _D), lambda i: (i, 0)),
            pl.BlockSpec((_D, _HID), lambda i: (0, 0)),
            pl.BlockSpec((_D, _HID), lambda i: (0, 0)),
            pl.BlockSpec((2 * _HID, _HID), lambda i: (0, 0)),
            pl.BlockSpec((2 * _HID, _HID), lambda i: (0, 0)),
            pl.BlockSpec((2 * _HID, 64), lambda i: (0, 0)),
            pl.BlockSpec((1, 64), lambda i: (0, 0)),
        ],
        out_specs=pl.BlockSpec((_B, 64), lambda i: (0, 0)),
        out_shape=jax.ShapeDtypeStruct((_B, 64), jnp.float32),
        scratch_shapes=[
            pltpu.VMEM((_B, 2 * _HID), jnp.float32),
            pltpu.VMEM((_B, _HID), jnp.float32),
        ],
    )(f0, f1, sum2, ws1, wn1, ws2, wn2, fcw, fcb)


def kernel(ids, feats, adj, W_self1, W_neib1, W_self2, W_neib2, fc_W, fc_b):
    ids = ids.astype(jnp.int32)
    adjf = adj.astype(jnp.int32).reshape(-1)
    r1 = jnp.asarray(_R1)
    s1 = jnp.asarray(_SEL1)
    r2 = jnp.asarray(_R2)
    s2 = jnp.asarray(_SEL2)
    f0, f1, sum2, _ = _sc_gather(ids, feats, adjf, r1, s1, r2, s2)
    return _tc_compute(f0, f1, sum2, W_self1, W_neib1, W_self2, W_neib2,
                       fc_W, fc_b.reshape(1, -1))


# speedup vs baseline: 6.6373x; 1.0032x over previous
import jax
import jax.numpy as jnp
import numpy as np
from jax import lax
from jax.experimental import pallas as pl
from jax.experimental.pallas import tpu as pltpu
from jax.experimental.pallas import tpu_sc as plsc

_N_NODES = 100000
_D = 128
_DEG = 32
_B = 1024
_NS1 = 25
_NS2 = 10
_HID = 128
_NW = 32
_SEEDS_W = _B // _NW
_CH = 80
_C1 = (_B * _NS1) // (_NW * _CH)
_C2 = (_B * _NS1 * _NS2) // (_NW * _CH)
_G1_W = _C1 * _CH
_G2_W = _C2 * _CH


def _rotl32(x, d):
    d = np.uint32(d)
    return (x << d) | (x >> np.uint32(32 - d))


def _tf2x32(k1, k2, c1, c2):
    rot = ((13, 15, 26, 6), (17, 29, 16, 24))
    ks = (k1, k2, k1 ^ k2 ^ np.uint32(0x1BD11BDA))
    x0 = (c1 + ks[0]).astype(np.uint32)
    x1 = (c2 + ks[1]).astype(np.uint32)
    for i in range(5):
        for r in rot[i % 2]:
            x0 = (x0 + x1).astype(np.uint32)
            x1 = _rotl32(x1, r)
            x1 = x0 ^ x1
        x0 = (x0 + ks[(i + 1) % 3]).astype(np.uint32)
        x1 = (x1 + ks[(i + 2) % 3] + np.uint32(i + 1)).astype(np.uint32)
    return x0, x1


def _sel_draw(layer, n):
    key = (np.zeros(1, np.uint32), np.full(1, 42, np.uint32))
    key = _tf2x32(key[0], key[1], np.zeros(1, np.uint32),
                  np.full(1, layer, np.uint32))
    b1, b2 = _tf2x32(key[0], key[1], np.zeros(2, np.uint32),
                     np.arange(2, dtype=np.uint32))
    k_hi = (b1[0:1], b2[0:1])
    k_lo = (b1[1:2], b2[1:2])
    idx = np.arange(n, dtype=np.uint64)
    c1 = (idx >> np.uint64(32)).astype(np.uint32)
    c2 = (idx & np.uint64(0xFFFFFFFF)).astype(np.uint32)
    h1, h2 = _tf2x32(k_hi[0], k_hi[1], c1, c2)
    l1, l2 = _tf2x32(k_lo[0], k_lo[1], c1, c2)
    hi = h1 ^ h2
    lo = l1 ^ l2
    return ((hi % np.uint32(_DEG)) * np.uint32(0)
            + (lo % np.uint32(_DEG))).astype(np.int32)


_SEL1 = _sel_draw(0, _B * _NS1).reshape(_NW, _C1, _CH)
_R1 = (np.arange(_B * _NS1, dtype=np.int32) // _NS1).reshape(_NW, _C1, _CH)


def _grp_transpose(t):
    return (t.reshape(_NW, _C2, _CH // _NS2, _NS2)
            .transpose(0, 1, 3, 2).reshape(_NW, _C2, _CH).copy())


_SEL2 = _grp_transpose(_sel_draw(1, _B * _NS1 * _NS2).reshape(_NW, _C2, _CH))
_R2 = _grp_transpose(
    (np.arange(_B * _NS1 * _NS2, dtype=np.int32) // _NS2).reshape(
        _NW, _C2, _CH))
_NSLOT = 6


def _sc_body(ids_h, feats_h, adjf_h, r1_h, s1_h, r2_h, s2_h,
             f0_h, f1_h, sum2_h, cur1_h,
             ids_v, bufa1, bufb1, cur1_v, bufa2, bufb2, cur2_v,
             fbuf, sbuf, accbuf, gsem, fsem, wsem):
    c = lax.axis_index("c")
    s = lax.axis_index("s")
    w = s * 2 + c
    seed0 = w * _SEEDS_W
    g0 = w * _G1_W
    ngrp = _CH // _NS2

    pltpu.sync_copy(ids_h.at[pl.ds(seed0, _SEEDS_W)], ids_v)
    pltpu.async_copy(feats_h.at[ids_v], sbuf, fsem)

    pltpu.sync_copy(r1_h.at[w], bufa1)

    def gi1(k, carry):
        pltpu.async_copy(ids_h.at[bufa1.at[k]], bufb1.at[k], gsem)
        return carry

    def di1(k, carry):
        pltpu.make_async_copy(ids_h.at[bufa1.at[k]], bufb1.at[k], gsem).wait()
        return carry

    lax.fori_loop(0, _C1, gi1, 0)
    lax.fori_loop(0, _C1, di1, 0)
    pltpu.sync_copy(s1_h.at[w], bufa1)

    def ma1(r, carry):
        for seg in range(_CH // 16):
            x = bufb1[r, pl.ds(seg * 16, 16)]
            y = bufa1[r, pl.ds(seg * 16, 16)]
            bufb1[r, pl.ds(seg * 16, 16)] = x * _DEG + y
        return carry

    lax.fori_loop(0, _C1, ma1, 0)

    def gc1(k, carry):
        pltpu.async_copy(adjf_h.at[bufb1.at[k]], cur1_v.at[k], gsem)
        return carry

    def dc1(k, carry):
        pltpu.make_async_copy(adjf_h.at[bufb1.at[k]], cur1_v.at[k],
                              gsem).wait()
        return carry

    lax.fori_loop(0, _C1, gc1, 0)
    lax.fori_loop(0, _C1, dc1, 0)

    def wc1(k, carry):
        pltpu.sync_copy(cur1_v.at[k], cur1_h.at[pl.ds(g0 + k * _CH, _CH)])
        return carry

    lax.fori_loop(0, _C1, wc1, 0)

    pltpu.sync_copy(r2_h.at[w], bufa2)

    def gi2(k, carry):
        pltpu.async_copy(cur1_h.at[bufa2.at[k]], bufb2.at[k], gsem)
        return carry

    lax.fori_loop(0, _C2, gi2, 0)

    pltpu.make_async_copy(feats_h.at[ids_v], sbuf, fsem).wait()
    pltpu.sync_copy(sbuf, f0_h.at[pl.ds(seed0, _SEEDS_W)])

    for p in range(_NSLOT - 1):
        pltpu.async_copy(feats_h.at[cur1_v.at[p]],
                         fbuf.at[pl.ds(p * _CH, _CH)], fsem)

    def f1loop(k, carry):
        off = (k % _NSLOT) * _CH
        pltpu.make_async_copy(feats_h.at[cur1_v.at[k]],
                              fbuf.at[pl.ds(off, _CH)], fsem).wait()

        @pl.when(k >= 1)
        def _drain_w():
            ko = k - 1
            pltpu.make_async_copy(
                fbuf.at[pl.ds((ko % _NSLOT) * _CH, _CH)],
                f1_h.at[pl.ds(g0 + ko * _CH, _CH)], wsem).wait()

        @pl.when(k < _C1 - (_NSLOT - 1))
        def _fire():
            kn = k + _NSLOT - 1
            pltpu.async_copy(feats_h.at[cur1_v.at[kn]],
                             fbuf.at[pl.ds((kn % _NSLOT) * _CH, _CH)], fsem)

        pltpu.async_copy(fbuf.at[pl.ds(off, _CH)],
                         f1_h.at[pl.ds(g0 + k * _CH, _CH)], wsem)
        return carry

    lax.fori_loop(0, _C1, f1loop, 0)
    ko = _C1 - 1
    pltpu.make_async_copy(fbuf.at[pl.ds((ko % _NSLOT) * _CH, _CH)],
                          f1_h.at[pl.ds(g0 + ko * _CH, _CH)], wsem).wait()

    def di2(k, carry):
        pltpu.make_async_copy(cur1_h.at[bufa2.at[k]], bufb2.at[k],
                              gsem).wait()
        return carry

    lax.fori_loop(0, _C2, di2, 0)
    pltpu.sync_copy(s2_h.at[w], bufa2)

    def ma2(r, carry):
        for seg in range(_CH // 16):
            x = bufb2[r, pl.ds(seg * 16, 16)]
            y = bufa2[r, pl.ds(seg * 16, 16)]
            bufb2[r, pl.ds(seg * 16, 16)] = x * _DEG + y
        return carry

    lax.fori_loop(0, _C2, ma2, 0)

    def gc2(k, carry):
        pltpu.async_copy(adjf_h.at[bufb2.at[k]], cur2_v.at[k], gsem)
        return carry

    def dc2(k, carry):
        pltpu.make_async_copy(adjf_h.at[bufb2.at[k]], cur2_v.at[k],
                              gsem).wait()
        return carry

    lax.fori_loop(0, _C2, gc2, 0)
    lax.fori_loop(0, _C2, dc2, 0)

    def l2fire(k, slot):
        for r in range(ngrp):
            for seg in range(_D // 16):
                accbuf[slot * ngrp + r, pl.ds(seg * 16, 16)] = jnp.zeros(
                    (16,), jnp.float32)
        for j in range(_NS2):
            pltpu.async_copy(
                feats_h.at[cur2_v.at[k, pl.ds(j * ngrp, ngrp)]],
                accbuf.at[pl.ds(slot * ngrp, ngrp)], fsem,
                add=True)

    def l2wait(k, slot):
        for j in range(_NS2):
            pltpu.make_async_copy(
                feats_h.at[cur2_v.at[k, pl.ds(j * ngrp, ngrp)]],
                accbuf.at[pl.ds(slot * ngrp, ngrp)], fsem).wait()

    for p in range(_NSLOT - 1):
        l2fire(p, p)

    def cc(k, carry):
        slot = k % _NSLOT
        l2wait(k, slot)

        @pl.when(k >= 1)
        def _drain_w():
            ko = k - 1
            pltpu.make_async_copy(
                accbuf.at[pl.ds((ko % _NSLOT) * ngrp, ngrp)],
                sum2_h.at[pl.ds(g0 + ko * ngrp, ngrp)], wsem).wait()

        @pl.when(k < _C2 - (_NSLOT - 1))
        def _fire():
            kn = k + _NSLOT - 1
            l2fire(kn, kn % _NSLOT)

        pltpu.async_copy(accbuf.at[pl.ds(slot * ngrp, ngrp)],
                         sum2_h.at[pl.ds(g0 + k * ngrp, ngrp)], wsem)
        return carry

    lax.fori_loop(0, _C2, cc, 0)
    ko = _C2 - 1
    pltpu.make_async_copy(accbuf.at[pl.ds((ko % _NSLOT) * ngrp, ngrp)],
                          sum2_h.at[pl.ds(g0 + ko * ngrp, ngrp)],
                          wsem).wait()


def _sc_gather(ids, feats, adjf, r1, s1, r2, s2):
    mesh = plsc.VectorSubcoreMesh(core_axis_name="c", subcore_axis_name="s")
    f = pl.kernel(
        _sc_body,
        out_type=(
            jax.ShapeDtypeStruct((_B, _D), jnp.float32),
            jax.ShapeDtypeStruct((_B * _NS1, _D), jnp.float32),
            jax.ShapeDtypeStruct((_B * _NS1, _D), jnp.float32),
            jax.ShapeDtypeStruct((_B * _NS1,), jnp.int32),
        ),
        mesh=mesh,
        scratch_types=[
            pltpu.VMEM((_SEEDS_W,), jnp.int32),
            pltpu.VMEM((_C1, _CH), jnp.int32),
            pltpu.VMEM((_C1, _CH), jnp.int32),
            pltpu.VMEM((_C1, _CH), jnp.int32),
            pltpu.VMEM((_C2, _CH), jnp.int32),
            pltpu.VMEM((_C2, _CH), jnp.int32),
            pltpu.VMEM((_C2, _CH), jnp.int32),
            pltpu.VMEM((_NSLOT * _CH, _D), jnp.float32),
            pltpu.VMEM((_SEEDS_W, _D), jnp.float32),
            pltpu.VMEM((_NSLOT * (_CH // _NS2), _D), jnp.float32),
            pltpu.SemaphoreType.DMA,
            pltpu.SemaphoreType.DMA,
            pltpu.SemaphoreType.DMA,
        ],
    )
    return f(ids, feats, adjf, r1, s1, r2, s2)


_RB = 1600
_GB = _RB // _NS1
_NSTEP = (_B * _NS1) // _RB


def _tc_body(f0_ref, f1_ref, sum2_ref, ws1_ref, wn1_ref, ws2_ref, wn2_ref,
             fcw_ref, fcb_ref, out_ref, acc_h1, acc_f1):
    i = pl.program_id(0)
    bf = jnp.bfloat16
    f1 = f1_ref[...].astype(bf)
    s2 = sum2_ref[...]
    ws1 = ws1_ref[...].astype(bf)
    wn1 = wn1_ref[...].astype(bf)
    a = jnp.maximum(jnp.dot(f1, ws1, preferred_element_type=jnp.float32), 0.0)
    b = jnp.maximum(jnp.dot((s2 * (1.0 / _NS2)).astype(bf), wn1,
                            preferred_element_type=jnp.float32), 0.0)
    h1 = jnp.concatenate([a, b], axis=1).astype(bf)
    rsel = lax.broadcasted_iota(jnp.int32, (_GB, _RB), 1) // _NS1
    gsel = (rsel == lax.broadcasted_iota(jnp.int32, (_GB, _RB), 0)).astype(bf)
    acc_h1[pl.ds(i * _GB, _GB), :] = jnp.dot(gsel, h1, preferred_element_type=jnp.float32)
    acc_f1[pl.ds(i * _GB, _GB), :] = jnp.dot(gsel, f1, preferred_element_type=jnp.float32)

    @pl.when(i == _NSTEP - 1)
    def _final():
        inv = 1.0 / _NS1
        f0 = f0_ref[...].astype(bf)
        h0a = jnp.maximum(jnp.dot(f0, ws1, preferred_element_type=jnp.float32), 0.0)
        h0b = jnp.maximum(jnp.dot((acc_f1[...] * inv).astype(bf), wn1,
                                  preferred_element_type=jnp.float32), 0.0)
        h0 = jnp.concatenate([h0a, h0b], axis=1).astype(bf)
        ha = jnp.maximum(jnp.dot(h0, ws2_ref[...].astype(bf),
                                 preferred_element_type=jnp.float32), 0.0)
        hb = jnp.maximum(jnp.dot((acc_h1[...] * inv).astype(bf),
                                 wn2_ref[...].astype(bf),
                                 preferred_element_type=jnp.float32), 0.0)
        hp = jnp.concatenate([ha, hb], axis=1)
        ss = jnp.sum(hp * hp, axis=1, keepdims=True)
        norm = jnp.maximum(jnp.sqrt(ss), 1e-12)
        out_ref[...] = (jnp.dot((hp / norm).astype(bf),
                                fcw_ref[...].astype(bf),
                                preferred_element_type=jnp.float32)
                        + fcb_ref[...])


def _tc_compute(f0, f1, sum2, ws1, wn1, ws2, wn2, fcw, fcb):
    return pl.pallas_call(
        _tc_body,
        grid=(_NSTEP,),
        in_specs=[
            pl.BlockSpec((_B, _D), lambda i: (0, 0)),
            pl.BlockSpec((_RB, _D), lambda i: (i, 0)),
            pl.BlockSpec((_RB, _D), lambda i: (i, 0)),
            pl.BlockSpec((_D, _HID), lambda i: (0, 0)),
            pl.BlockSpec((_D, _HID), lambda i: (0, 0)),
            pl.BlockSpec((2 * _HID, _HID), lambda i: (0, 0)),
            pl.BlockSpec((2 * _HID, _HID), lambda i: (0, 0)),
            pl.BlockSpec((2 * _HID, 64), lambda i: (0, 0)),
            pl.BlockSpec((1, 64), lambda i: (0, 0)),
        ],
        out_specs=pl.BlockSpec((_B, 64), lambda i: (0, 0)),
        out_shape=jax.ShapeDtypeStruct((_B, 64), jnp.float32),
        scratch_shapes=[
            pltpu.VMEM((_B, 2 * _HID), jnp.float32),
            pltpu.VMEM((_B, _HID), jnp.float32),
        ],
    )(f0, f1, sum2, ws1, wn1, ws2, wn2, fcw, fcb)


def kernel(ids, feats, adj, W_self1, W_neib1, W_self2, W_neib2, fc_W, fc_b):
    ids = ids.astype(jnp.int32)
    adjf = adj.astype(jnp.int32).reshape(-1)
    r1 = jnp.asarray(_R1)
    s1 = jnp.asarray(_SEL1)
    r2 = jnp.asarray(_R2)
    s2 = jnp.asarray(_SEL2)
    f0, f1, sum2, _ = _sc_gather(ids, feats, adjf, r1, s1, r2, s2)
    return _tc_compute(f0, f1, sum2, W_self1, W_neib1, W_self2, W_neib2,
                       fc_W, fc_b.reshape(1, -1))


# speedup vs baseline: 8.1200x; 1.2234x over previous
import jax
import jax.numpy as jnp
import numpy as np
from jax import lax
from jax.experimental import pallas as pl
from jax.experimental.pallas import tpu as pltpu
from jax.experimental.pallas import tpu_sc as plsc

_N_NODES = 100000
_D = 128
_DEG = 32
_B = 1024
_NS1 = 25
_NS2 = 10
_HID = 128
_NW = 32
_SEEDS_W = _B // _NW
_CH = 80
_C1 = (_B * _NS1) // (_NW * _CH)
_G1_W = _C1 * _CH
_G2_W = _G1_W * _NS2
_VL = 16
_CH2 = _VL * _NS2
_C2 = _G2_W // _CH2


def _rotl32(x, d):
    d = np.uint32(d)
    return (x << d) | (x >> np.uint32(32 - d))


def _tf2x32(k1, k2, c1, c2):
    rot = ((13, 15, 26, 6), (17, 29, 16, 24))
    ks = (k1, k2, k1 ^ k2 ^ np.uint32(0x1BD11BDA))
    x0 = (c1 + ks[0]).astype(np.uint32)
    x1 = (c2 + ks[1]).astype(np.uint32)
    for i in range(5):
        for r in rot[i % 2]:
            x0 = (x0 + x1).astype(np.uint32)
            x1 = _rotl32(x1, r)
            x1 = x0 ^ x1
        x0 = (x0 + ks[(i + 1) % 3]).astype(np.uint32)
        x1 = (x1 + ks[(i + 2) % 3] + np.uint32(i + 1)).astype(np.uint32)
    return x0, x1


def _sel_draw(layer, n):
    key = (np.zeros(1, np.uint32), np.full(1, 42, np.uint32))
    key = _tf2x32(key[0], key[1], np.zeros(1, np.uint32),
                  np.full(1, layer, np.uint32))
    b1, b2 = _tf2x32(key[0], key[1], np.zeros(2, np.uint32),
                     np.arange(2, dtype=np.uint32))
    k_hi = (b1[0:1], b2[0:1])
    k_lo = (b1[1:2], b2[1:2])
    idx = np.arange(n, dtype=np.uint64)
    c1 = (idx >> np.uint64(32)).astype(np.uint32)
    c2 = (idx & np.uint64(0xFFFFFFFF)).astype(np.uint32)
    h1, h2 = _tf2x32(k_hi[0], k_hi[1], c1, c2)
    l1, l2 = _tf2x32(k_lo[0], k_lo[1], c1, c2)
    hi = h1 ^ h2
    lo = l1 ^ l2
    return ((hi % np.uint32(_DEG)) * np.uint32(0)
            + (lo % np.uint32(_DEG))).astype(np.int32)


_SEL1 = _sel_draw(0, _B * _NS1).reshape(_NW, _C1, _CH)
_R1 = (np.arange(_B * _NS1, dtype=np.int32) // _NS1).reshape(_NW, _C1, _CH)


def _grp_transpose(t):
    return (t.reshape(_NW, _C2, _VL, _NS2)
            .transpose(0, 1, 3, 2).reshape(_NW, _C2, _CH2).copy())


_SEL2 = _grp_transpose(_sel_draw(1, _B * _NS1 * _NS2).reshape(_NW, _C2, _CH2))
_NSLOT = 6


def _sc_body(ids_h, feats_h, adjf_h, r1_h, s1_h, s2_h,
             f0_h, f1_h, sum2_h,
             ids_v, bufa1, bufb1, cur1_v, bufa2, bufb2, cur2_v,
             fbuf, sbuf, accbuf, gsem, fsem, wsem):
    c = lax.axis_index("c")
    s = lax.axis_index("s")
    w = s * 2 + c
    seed0 = w * _SEEDS_W
    g0 = w * _G1_W

    pltpu.sync_copy(ids_h.at[pl.ds(seed0, _SEEDS_W)], ids_v)
    pltpu.async_copy(feats_h.at[ids_v], sbuf, fsem)

    pltpu.sync_copy(r1_h.at[w], bufa1)

    def gi1(k, carry):
        pltpu.async_copy(ids_h.at[bufa1.at[k]], bufb1.at[k], gsem)
        return carry

    def di1(k, carry):
        pltpu.make_async_copy(ids_h.at[bufa1.at[k]], bufb1.at[k], gsem).wait()
        return carry

    lax.fori_loop(0, _C1, gi1, 0)
    lax.fori_loop(0, _C1, di1, 0)
    pltpu.sync_copy(s1_h.at[w], bufa1)

    def ma1(r, carry):
        for seg in range(_CH // 16):
            x = bufb1[r, pl.ds(seg * 16, 16)]
            y = bufa1[r, pl.ds(seg * 16, 16)]
            bufb1[r, pl.ds(seg * 16, 16)] = x * _DEG + y
        return carry

    lax.fori_loop(0, _C1, ma1, 0)

    def gc1(k, carry):
        pltpu.async_copy(adjf_h.at[bufb1.at[k]], cur1_v.at[k], gsem)
        return carry

    def dc1(k, carry):
        pltpu.make_async_copy(adjf_h.at[bufb1.at[k]], cur1_v.at[k],
                              gsem).wait()
        return carry

    lax.fori_loop(0, _C1, gc1, 0)
    lax.fori_loop(0, _C1, dc1, 0)

    pltpu.sync_copy(s2_h.at[w], bufa2)

    def mkidx(r, carry):
        for t in range(_CH // _VL):
            k = r * (_CH // _VL) + t
            v = cur1_v[r, pl.ds(t * _VL, _VL)] * _DEG
            for j in range(_NS2):
                bufb2[pl.ds(k * _CH2 + j * _VL, _VL)] = (
                    v + bufa2[k, pl.ds(j * _VL, _VL)])
            pltpu.async_copy(adjf_h.at[bufb2.at[pl.ds(k * _CH2, _CH2)]],
                             cur2_v.at[pl.ds(k * _CH2, _CH2)], gsem)
        return carry

    lax.fori_loop(0, _C1, mkidx, 0)

    pltpu.make_async_copy(feats_h.at[ids_v], sbuf, fsem).wait()
    pltpu.sync_copy(sbuf, f0_h.at[pl.ds(seed0, _SEEDS_W)])

    for p in range(_NSLOT - 1):
        pltpu.async_copy(feats_h.at[cur1_v.at[p]],
                         fbuf.at[pl.ds(p * _CH, _CH)], fsem)

    def f1loop(k, carry):
        off = (k % _NSLOT) * _CH
        pltpu.make_async_copy(feats_h.at[cur1_v.at[k]],
                              fbuf.at[pl.ds(off, _CH)], fsem).wait()

        @pl.when(k >= 1)
        def _drain_w():
            ko = k - 1
            pltpu.make_async_copy(
                fbuf.at[pl.ds((ko % _NSLOT) * _CH, _CH)],
                f1_h.at[pl.ds(g0 + ko * _CH, _CH)], wsem).wait()

        @pl.when(k < _C1 - (_NSLOT - 1))
        def _fire():
            kn = k + _NSLOT - 1
            pltpu.async_copy(feats_h.at[cur1_v.at[kn]],
                             fbuf.at[pl.ds((kn % _NSLOT) * _CH, _CH)], fsem)

        pltpu.async_copy(fbuf.at[pl.ds(off, _CH)],
                         f1_h.at[pl.ds(g0 + k * _CH, _CH)], wsem)
        return carry

    lax.fori_loop(0, _C1, f1loop, 0)
    ko = _C1 - 1
    pltpu.make_async_copy(fbuf.at[pl.ds((ko % _NSLOT) * _CH, _CH)],
                          f1_h.at[pl.ds(g0 + ko * _CH, _CH)], wsem).wait()

    def dc2(k, carry):
        pltpu.make_async_copy(adjf_h.at[bufb2.at[pl.ds(k * _CH2, _CH2)]],
                              cur2_v.at[pl.ds(k * _CH2, _CH2)], gsem).wait()
        return carry

    lax.fori_loop(0, _C2, dc2, 0)

    def l2fire(k, slot):
        for r in range(_VL):
            for seg in range(_D // 16):
                accbuf[slot * _VL + r, pl.ds(seg * 16, 16)] = jnp.zeros(
                    (16,), jnp.float32)
        for j in range(_NS2):
            pltpu.async_copy(
                feats_h.at[cur2_v.at[pl.ds(k * _CH2 + j * _VL, _VL)]],
                accbuf.at[pl.ds(slot * _VL, _VL)], fsem,
                add=True)

    def l2wait(k, slot):
        for j in range(_NS2):
            pltpu.make_async_copy(
                feats_h.at[cur2_v.at[pl.ds(k * _CH2 + j * _VL, _VL)]],
                accbuf.at[pl.ds(slot * _VL, _VL)], fsem).wait()

    for p in range(_NSLOT - 1):
        l2fire(p, p)

    def cc(k, carry):
        slot = k % _NSLOT
        l2wait(k, slot)

        @pl.when(k >= 1)
        def _drain_w():
            ko = k - 1
            pltpu.make_async_copy(
                accbuf.at[pl.ds((ko % _NSLOT) * _VL, _VL)],
                sum2_h.at[pl.ds(g0 + ko * _VL, _VL)], wsem).wait()

        @pl.when(k < _C2 - (_NSLOT - 1))
        def _fire():
            kn = k + _NSLOT - 1
            l2fire(kn, kn % _NSLOT)

        pltpu.async_copy(accbuf.at[pl.ds(slot * _VL, _VL)],
                         sum2_h.at[pl.ds(g0 + k * _VL, _VL)], wsem)
        return carry

    lax.fori_loop(0, _C2, cc, 0)
    ko = _C2 - 1
    pltpu.make_async_copy(accbuf.at[pl.ds((ko % _NSLOT) * _VL, _VL)],
                          sum2_h.at[pl.ds(g0 + ko * _VL, _VL)],
                          wsem).wait()


def _sc_gather(ids, feats, adjf, r1, s1, s2):
    mesh = plsc.VectorSubcoreMesh(core_axis_name="c", subcore_axis_name="s")
    f = pl.kernel(
        _sc_body,
        out_type=(
            jax.ShapeDtypeStruct((_B, _D), jnp.float32),
            jax.ShapeDtypeStruct((_B * _NS1, _D), jnp.float32),
            jax.ShapeDtypeStruct((_B * _NS1, _D), jnp.float32),
        ),
        mesh=mesh,
        scratch_types=[
            pltpu.VMEM((_SEEDS_W,), jnp.int32),
            pltpu.VMEM((_C1, _CH), jnp.int32),
            pltpu.VMEM((_C1, _CH), jnp.int32),
            pltpu.VMEM((_C1, _CH), jnp.int32),
            pltpu.VMEM((_C2, _CH2), jnp.int32),
            pltpu.VMEM((_C2 * _CH2,), jnp.int32),
            pltpu.VMEM((_C2 * _CH2,), jnp.int32),
            pltpu.VMEM((_NSLOT * _CH, _D), jnp.float32),
            pltpu.VMEM((_SEEDS_W, _D), jnp.float32),
            pltpu.VMEM((_NSLOT * _VL, _D), jnp.float32),
            pltpu.SemaphoreType.DMA,
            pltpu.SemaphoreType.DMA,
            pltpu.SemaphoreType.DMA,
        ],
    )
    return f(ids, feats, adjf, r1, s1, s2)


_RB = 1600
_GB = _RB // _NS1
_NSTEP = (_B * _NS1) // _RB


def _tc_body(f0_ref, f1_ref, sum2_ref, ws1_ref, wn1_ref, ws2_ref, wn2_ref,
             fcw_ref, fcb_ref, out_ref, acc_h1, acc_f1):
    i = pl.program_id(0)
    bf = jnp.bfloat16
    f1 = f1_ref[...].astype(bf)
    s2 = sum2_ref[...]
    ws1 = ws1_ref[...].astype(bf)
    wn1 = wn1_ref[...].astype(bf)
    a = jnp.maximum(jnp.dot(f1, ws1, preferred_element_type=jnp.float32), 0.0)
    b = jnp.maximum(jnp.dot((s2 * (1.0 / _NS2)).astype(bf), wn1,
                            preferred_element_type=jnp.float32), 0.0)
    h1 = jnp.concatenate([a, b], axis=1).astype(bf)
    rsel = lax.broadcasted_iota(jnp.int32, (_GB, _RB), 1) // _NS1
    gsel = (rsel == lax.broadcasted_iota(jnp.int32, (_GB, _RB), 0)).astype(bf)
    acc_h1[pl.ds(i * _GB, _GB), :] = jnp.dot(gsel, h1, preferred_element_type=jnp.float32)
    acc_f1[pl.ds(i * _GB, _GB), :] = jnp.dot(gsel, f1, preferred_element_type=jnp.float32)

    @pl.when(i == _NSTEP - 1)
    def _final():
        inv = 1.0 / _NS1
        f0 = f0_ref[...].astype(bf)
        h0a = jnp.maximum(jnp.dot(f0, ws1, preferred_element_type=jnp.float32), 0.0)
        h0b = jnp.maximum(jnp.dot((acc_f1[...] * inv).astype(bf), wn1,
                                  preferred_element_type=jnp.float32), 0.0)
        h0 = jnp.concatenate([h0a, h0b], axis=1).astype(bf)
        ha = jnp.maximum(jnp.dot(h0, ws2_ref[...].astype(bf),
                                 preferred_element_type=jnp.float32), 0.0)
        hb = jnp.maximum(jnp.dot((acc_h1[...] * inv).astype(bf),
                                 wn2_ref[...].astype(bf),
                                 preferred_element_type=jnp.float32), 0.0)
        hp = jnp.concatenate([ha, hb], axis=1)
        ss = jnp.sum(hp * hp, axis=1, keepdims=True)
        norm = jnp.maximum(jnp.sqrt(ss), 1e-12)
        out_ref[...] = (jnp.dot((hp / norm).astype(bf),
                                fcw_ref[...].astype(bf),
                                preferred_element_type=jnp.float32)
                        + fcb_ref[...])


def _tc_compute(f0, f1, sum2, ws1, wn1, ws2, wn2, fcw, fcb):
    return pl.pallas_call(
        _tc_body,
        grid=(_NSTEP,),
        in_specs=[
            pl.BlockSpec((_B, _D), lambda i: (0, 0)),
            pl.BlockSpec((_RB, _D), lambda i: (i, 0)),
            pl.BlockSpec((_RB, _D), lambda i: (i, 0)),
            pl.BlockSpec((_D, _HID), lambda i: (0, 0)),
            pl.BlockSpec((_D, _HID), lambda i: (0, 0)),
            pl.BlockSpec((2 * _HID, _HID), lambda i: (0, 0)),
            pl.BlockSpec((2 * _HID, _HID), lambda i: (0, 0)),
            pl.BlockSpec((2 * _HID, 64), lambda i: (0, 0)),
            pl.BlockSpec((1, 64), lambda i: (0, 0)),
        ],
        out_specs=pl.BlockSpec((_B, 64), lambda i: (0, 0)),
        out_shape=jax.ShapeDtypeStruct((_B, 64), jnp.float32),
        scratch_shapes=[
            pltpu.VMEM((_B, 2 * _HID), jnp.float32),
            pltpu.VMEM((_B, _HID), jnp.float32),
        ],
    )(f0, f1, sum2, ws1, wn1, ws2, wn2, fcw, fcb)


def kernel(ids, feats, adj, W_self1, W_neib1, W_self2, W_neib2, fc_W, fc_b):
    ids = ids.astype(jnp.int32)
    adjf = adj.astype(jnp.int32).reshape(-1)
    r1 = jnp.asarray(_R1)
    s1 = jnp.asarray(_SEL1)
    s2 = jnp.asarray(_SEL2)
    f0, f1, sum2 = _sc_gather(ids, feats, adjf, r1, s1, s2)
    return _tc_compute(f0, f1, sum2, W_self1, W_neib1, W_self2, W_neib2,
                       fc_W, fc_b.reshape(1, -1))
